# baseline (speedup 1.0000x reference)
import functools

import jax
import jax.numpy as jnp
from jax import lax
from jax.experimental import pallas as pl
from jax.experimental.pallas import tpu as pltpu
from jax.experimental.pallas import tpu_sc as plsc

SHIFT_AMT = 8192
TIME = 441000
LENGTH = TIME - SHIFT_AMT
ROWS = 128
NWORKERS = 32
R_SC = 32
R_TC = ROWS - R_SC
ROWS_PER_W = R_SC // NWORKERS
CHUNK = 16128
NFULL = LENGTH // CHUNK
REM = LENGTH - NFULL * CHUNK
NCH = NFULL + 1
UNROLL = 8
NBUF = 8
DEPTH = NBUF - 2


@functools.partial(
    pl.kernel,
    out_type=jax.ShapeDtypeStruct((R_SC * LENGTH,), jnp.float32),
    mesh=plsc.VectorSubcoreMesh(core_axis_name="c", subcore_axis_name="s"),
    scratch_types=[pltpu.VMEM((NWORKERS * 16,), jnp.int32)]
    + [pltpu.VMEM((CHUNK + 8,), jnp.float32) for _ in range(NBUF)]
    + [pltpu.SemaphoreType.DMA for _ in range(2 * NBUF)],
)
def _shift_sc(wav_hbm, offs_hbm, out_hbm, offs_v, *bufs_and_sems):
    bufs = bufs_and_sems[:NBUF]
    in_sems = bufs_and_sems[NBUF:2 * NBUF]
    out_sems = bufs_and_sems[2 * NBUF:]

    wid = lax.axis_index("s") * 2 + lax.axis_index("c")
    pltpu.sync_copy(offs_hbm, offs_v)
    vec = offs_v[pl.ds(pl.multiple_of(wid * 16, 16), 16)]

    units = []
    for j in range(ROWS_PER_W):
        row = wid * ROWS_PER_W + j
        off = vec[j]
        r = off & 7
        src0 = pl.multiple_of(row * TIME + (off & ~7), 8)
        dst0 = pl.multiple_of(row * LENGTH, 8)
        for t in range(NCH):
            n = CHUNK if t < NFULL else REM
            trips = jnp.where(r == 0, 0, (n + 16 * UNROLL - 1) // (16 * UNROLL))
            units.append((
                pl.multiple_of(src0 + t * CHUNK, 8),
                pl.multiple_of(dst0 + t * CHUNK, 8),
                n, r, trips,
            ))
    nu = len(units)

    def issue_in(u):
        src, _, n, _, _ = units[u]
        b = u % NBUF
        return pltpu.async_copy(
            wav_hbm.at[pl.ds(src, n + 8)], bufs[b].at[pl.ds(0, n + 8)],
            in_sems[b],
        )

    def issue_out(u):
        _, dst, n, _, _ = units[u]
        b = u % NBUF
        return pltpu.async_copy(
            bufs[b].at[pl.ds(0, n)], out_hbm.at[pl.ds(dst, n)], out_sems[b],
        )

    in_copies = [None] * nu
    out_copies = [None] * nu
    for u in range(min(DEPTH, nu)):
        in_copies[u] = issue_in(u)
    for u in range(nu):
        if u + DEPTH < nu:
            if u - (NBUF - DEPTH) >= 0:
                out_copies[u - (NBUF - DEPTH)].wait()
            in_copies[u + DEPTH] = issue_in(u + DEPTH)
        in_copies[u].wait()
        _, _, n, r, trips = units[u]
        buf = bufs[u % NBUF]

        def realign(i, _, buf=buf, r=r):
            base = pl.multiple_of(i * (16 * UNROLL), 16)
            for k in range(UNROLL):
                buf[pl.ds(pl.multiple_of(base + k * 16, 16), 16)] = (
                    buf[pl.ds(base + k * 16 + r, 16)]
                )
            return 0

        lax.fori_loop(0, trips, realign, 0)
        out_copies[u] = issue_out(u)
    for u in range(max(0, nu - NBUF), nu):
        if out_copies[u] is not None:
            out_copies[u].wait()


TC_GROUP = 8
TC_STEPS = R_TC // TC_GROUP
SZIN = 433024
TOTAL = ROWS * TIME


def _tc_body(offs_ref, wav_hbm, out_hbm, *bufs_and_sems):
    ibufs = bufs_and_sems[:TC_GROUP]
    obuf = bufs_and_sems[TC_GROUP]
    isems = bufs_and_sems[TC_GROUP + 1:2 * TC_GROUP + 1]
    osem = bufs_and_sems[2 * TC_GROUP + 1]
    g = pl.program_id(0)
    row0 = g * TC_GROUP + R_SC
    copies = []
    shifts = []
    for i in range(TC_GROUP):
        row = row0 + i
        off = offs_ref[row]
        s = row * TIME + off
        src = pl.multiple_of(
            jnp.minimum(s & ~127, TOTAL - SZIN), 128
        )
        shifts.append(s - src)
        cp = pltpu.make_async_copy(
            wav_hbm.at[pl.ds(src, SZIN)], ibufs[i].at[pl.ds(0, SZIN)], isems[i]
        )
        cp.start()
        copies.append(cp)
    nrow = SZIN // 128
    lane = lax.broadcasted_iota(jnp.int32, (nrow - 1, 128), 1)
    for i in range(TC_GROUP):
        copies[i].wait()
        coarse = pl.multiple_of(shifts[i] & ~127, 128)
        fine = shifts[i] & 127
        v = ibufs[i][pl.ds(coarse, SZIN)].reshape(nrow, 128)
        rot = pltpu.roll(v, -fine, 1)
        hi = rot[: nrow - 1, :]
        lo = rot[1:nrow, :]
        sel = jnp.where(lane < 128 - fine, hi, lo)
        obuf[i, :] = sel.reshape((nrow - 1) * 128)[:LENGTH]
    ocp = pltpu.make_async_copy(
        obuf,
        out_hbm.at[pl.ds(pl.multiple_of(g * TC_GROUP, 8), TC_GROUP), :],
        osem,
    )
    ocp.start()
    ocp.wait()


def _shift_tc(wav1, offs_tc):
    grid_spec = pltpu.PrefetchScalarGridSpec(
        num_scalar_prefetch=1,
        grid=(TC_STEPS,),
        in_specs=[pl.BlockSpec(memory_space=pl.ANY)],
        out_specs=pl.BlockSpec(memory_space=pl.ANY),
        scratch_shapes=[
            pltpu.VMEM((SZIN + 128,), jnp.float32) for _ in range(TC_GROUP)
        ]
        + [pltpu.VMEM((TC_GROUP, LENGTH), jnp.float32)]
        + [pltpu.SemaphoreType.DMA for _ in range(TC_GROUP)]
        + [pltpu.SemaphoreType.DMA],
    )
    return pl.pallas_call(
        _tc_body,
        grid_spec=grid_spec,
        out_shape=jax.ShapeDtypeStruct((R_TC, LENGTH), jnp.float32),
    )(offs_tc, wav1)


def kernel(wav, offsets):
    batch, sources, channels, time = wav.shape
    wav1 = wav.reshape(ROWS * TIME)
    offs = jnp.broadcast_to(
        offsets.reshape(batch * sources, 1), (batch * sources, channels)
    ).reshape(ROWS)
    offs_pad = jnp.pad(
        offs[:R_SC].reshape(NWORKERS, ROWS_PER_W),
        ((0, 0), (0, 16 - ROWS_PER_W)),
    ).reshape(NWORKERS * 16)
    out_sc = _shift_sc(wav1, offs_pad)
    out_tc = _shift_tc(wav1, offs)
    out = jnp.concatenate(
        [out_sc.reshape(R_SC, LENGTH), out_tc], axis=0
    )
    return out.reshape(batch, sources, channels, LENGTH)
